# baseline (speedup 1.0000x reference)
import functools

import jax
import jax.numpy as jnp
from jax import lax
from jax.experimental import pallas as pl
from jax.experimental.pallas import tpu as pltpu
from jax.experimental.pallas import tpu_sc as plsc

B = 16384
D = 32
NV = 1000000
L = 16
NC, NS = 2, 16
NW = NC * NS
BW = B // NW
CH = 128
NCH = BW // CH


def _sc_predict(uid, iid, uemb_flat, iemb_flat, ub_flat, ib_flat, gb_splat):
    mesh = plsc.VectorSubcoreMesh(core_axis_name="c", subcore_axis_name="s")
    cp = pltpu.CompilerParams(
        needs_layout_passes=False,
        use_tc_tiling_on_sc=False,
    )

    @functools.partial(
        pl.kernel,
        out_type=jax.ShapeDtypeStruct((B,), jnp.float32),
        mesh=mesh,
        compiler_params=cp,
        scratch_types=[
            pltpu.VMEM((BW,), jnp.int32),
            pltpu.VMEM((BW,), jnp.int32),
            pltpu.VMEM((D * NCH, CH), jnp.int32),
            pltpu.VMEM((D * NCH, CH), jnp.int32),
            pltpu.VMEM((D, BW), jnp.float32),
            pltpu.VMEM((D, BW), jnp.float32),
            pltpu.VMEM((BW,), jnp.float32),
            pltpu.VMEM((BW,), jnp.float32),
            pltpu.VMEM((BW,), jnp.float32),
            pltpu.VMEM((L,), jnp.float32),
            pltpu.SemaphoreType.DMA,
            pltpu.SemaphoreType.DMA,
            pltpu.SemaphoreType.DMA,
            pltpu.SemaphoreType.DMA,
        ],
    )
    def k(uid_hbm, iid_hbm, uemb_hbm, iemb_hbm, ub_hbm, ib_hbm, gb_hbm,
          out_hbm, uidx, iidx, udx, idx2, uf, itf, ubv, ibv, outv, gbv,
          sem_u, sem_i, sem_ub, sem_ib):
        wid = lax.axis_index("s") * NC + lax.axis_index("c")
        base = wid * BW
        pltpu.sync_copy(uid_hbm.at[pl.ds(base, BW)], uidx)
        pltpu.sync_copy(iid_hbm.at[pl.ds(base, BW)], iidx)
        pltpu.sync_copy(gb_hbm, gbv)

        copies = []
        for c in range(NCH):
            dst = pl.ds(c * CH, CH)
            copies.append(pltpu.async_copy(
                ub_hbm.at[uidx.at[pl.ds(c * CH, CH)]], ubv.at[dst], sem_ub))
            copies.append(pltpu.async_copy(
                ib_hbm.at[iidx.at[pl.ds(c * CH, CH)]], ibv.at[dst], sem_ib))

        @pl.loop(0, D)
        def _(d):
            off = d * NV

            @pl.loop(0, BW, step=L)
            def _(t):
                row = d * NCH + t // CH
                j = t % CH
                udx[row, pl.ds(j, L)] = uidx[pl.ds(t, L)] + off
                idx2[row, pl.ds(j, L)] = iidx[pl.ds(t, L)] + off

        for d in range(D):
            for c in range(NCH):
                dst = pl.ds(c * CH, CH)
                copies.append(pltpu.async_copy(
                    uemb_hbm.at[udx.at[d * NCH + c]], uf.at[d, dst], sem_u))
                copies.append(pltpu.async_copy(
                    iemb_hbm.at[idx2.at[d * NCH + c]], itf.at[d, dst], sem_i))
        for cpy in copies:
            cpy.wait()

        gb = gbv[...]

        @pl.loop(0, BW, step=L)
        def _(t):
            s = pl.ds(t, L)
            acc = gb + ubv[s] + ibv[s]
            for d in range(D):
                acc = acc + uf[d, s] * itf[d, s]
            outv[s] = acc

        pltpu.sync_copy(outv, out_hbm.at[pl.ds(base, BW)])

    return k(uid, iid, uemb_flat, iemb_flat, ub_flat, ib_flat, gb_splat)


def kernel(user_ids, item_ids, user_emb, item_emb, user_bias, item_bias,
           global_bias):
    ufl = user_emb.T.reshape(-1)
    ifl = item_emb.T.reshape(-1)
    ub = user_bias.reshape(-1)
    ib = item_bias.reshape(-1)
    gbl = jnp.broadcast_to(global_bias.reshape(()), (L,))
    return _sc_predict(user_ids, item_ids, ufl, ifl, ub, ib, gbl)


# speedup vs baseline: 1.0017x; 1.0017x over previous
import functools

import jax
import jax.numpy as jnp
from jax import lax
from jax.experimental import pallas as pl
from jax.experimental.pallas import tpu as pltpu
from jax.experimental.pallas import tpu_sc as plsc

B = 16384
D = 32
NV = 1000000
L = 16
NC, NS = 2, 16
NW = NC * NS
BW = B // NW
CH = 128
NCH = BW // CH


def _sc_predict(uid, iid, uemb_flat, iemb_flat, ub_flat, ib_flat, gb_splat):
    mesh = plsc.VectorSubcoreMesh(core_axis_name="c", subcore_axis_name="s")
    cp = pltpu.CompilerParams(
        needs_layout_passes=False,
        use_tc_tiling_on_sc=False,
    )

    @functools.partial(
        pl.kernel,
        out_type=jax.ShapeDtypeStruct((B,), jnp.float32),
        mesh=mesh,
        compiler_params=cp,
        scratch_types=[
            pltpu.VMEM((NCH, CH), jnp.int32),
            pltpu.VMEM((NCH, CH), jnp.int32),
            pltpu.VMEM((D, BW), jnp.float32),
            pltpu.VMEM((D, BW), jnp.float32),
            pltpu.VMEM((BW,), jnp.float32),
            pltpu.VMEM((BW,), jnp.float32),
            pltpu.VMEM((BW,), jnp.float32),
            pltpu.VMEM((L,), jnp.float32),
            pltpu.SemaphoreType.DMA,
            pltpu.SemaphoreType.DMA,
            pltpu.SemaphoreType.DMA,
            pltpu.SemaphoreType.DMA,
        ],
    )
    def k(uid_hbm, iid_hbm, uemb_hbm, iemb_hbm, ub_hbm, ib_hbm, gb_hbm,
          out_hbm, uidx, iidx, uf, itf, ubv, ibv, outv, gbv,
          sem_u, sem_i, sem_ub, sem_ib):
        wid = lax.axis_index("s") * NC + lax.axis_index("c")
        base = wid * BW
        pltpu.sync_copy(uid_hbm.at[wid], uidx)
        pltpu.sync_copy(iid_hbm.at[wid], iidx)
        pltpu.sync_copy(gb_hbm, gbv)

        copies = []
        for c in range(NCH):
            dst = pl.ds(c * CH, CH)
            copies.append(pltpu.async_copy(
                ub_hbm.at[uidx.at[c]], ubv.at[dst], sem_ub))
            copies.append(pltpu.async_copy(
                ib_hbm.at[iidx.at[c]], ibv.at[dst], sem_ib))

        for d in range(D):
            for c in range(NCH):
                dst = pl.ds(c * CH, CH)
                copies.append(pltpu.async_copy(
                    uemb_hbm.at[d].at[uidx.at[c]], uf.at[d, dst], sem_u))
                copies.append(pltpu.async_copy(
                    iemb_hbm.at[d].at[iidx.at[c]], itf.at[d, dst], sem_i))
        for cpy in copies:
            cpy.wait()

        gb = gbv[...]

        @pl.loop(0, BW, step=L)
        def _(t):
            s = pl.ds(t, L)
            acc = gb + ubv[s] + ibv[s]
            for d in range(D):
                acc = acc + uf[d, s] * itf[d, s]
            outv[s] = acc

        pltpu.sync_copy(outv, out_hbm.at[pl.ds(base, BW)])

    return k(uid, iid, uemb_flat, iemb_flat, ub_flat, ib_flat, gb_splat)


def kernel(user_ids, item_ids, user_emb, item_emb, user_bias, item_bias,
           global_bias):
    uid3 = user_ids.reshape(NW, NCH, CH)
    iid3 = item_ids.reshape(NW, NCH, CH)
    ufl = user_emb.T
    ifl = item_emb.T
    ub = user_bias.reshape(-1)
    ib = item_bias.reshape(-1)
    gbl = jnp.broadcast_to(global_bias.reshape(()), (L,))
    return _sc_predict(uid3, iid3, ufl, ifl, ub, ib, gbl)


# speedup vs baseline: 5.8271x; 5.8172x over previous
import functools

import jax
import jax.numpy as jnp
from jax import lax
from jax.experimental import pallas as pl
from jax.experimental.pallas import tpu as pltpu
from jax.experimental.pallas import tpu_sc as plsc

B = 16384
D = 32
L = 16
NC, NS = 2, 16
NW = NC * NS
BW = B // NW
CH = 128
NCH = BW // CH


def _sc_predict(uid, iid, user_emb, item_emb, ub_flat, ib_flat, global_bias):
    mesh = plsc.VectorSubcoreMesh(core_axis_name="c", subcore_axis_name="s")
    cp = pltpu.CompilerParams(
        needs_layout_passes=False,
        use_tc_tiling_on_sc=False,
    )

    @functools.partial(
        pl.kernel,
        out_type=jax.ShapeDtypeStruct((B,), jnp.float32),
        mesh=mesh,
        compiler_params=cp,
        scratch_types=[
            pltpu.VMEM((NCH, CH), jnp.int32),
            pltpu.VMEM((NCH, CH), jnp.int32),
            pltpu.VMEM((BW, D), jnp.float32),
            pltpu.VMEM((BW, D), jnp.float32),
            pltpu.VMEM((BW,), jnp.float32),
            pltpu.VMEM((BW,), jnp.float32),
            pltpu.VMEM((BW,), jnp.float32),
            pltpu.VMEM((L,), jnp.float32),
            pltpu.SemaphoreType.DMA,
            pltpu.SemaphoreType.DMA,
            pltpu.SemaphoreType.DMA,
            pltpu.SemaphoreType.DMA,
        ],
    )
    def k(uid_hbm, iid_hbm, uemb_hbm, iemb_hbm, ub_hbm, ib_hbm, gb_hbm,
          out_hbm, uidx, iidx, urows, irows, ubv, ibv, outv, gbv,
          sem_u, sem_i, sem_ub, sem_ib):
        wid = lax.axis_index("s") * NC + lax.axis_index("c")
        pltpu.sync_copy(uid_hbm.at[wid], uidx)
        pltpu.sync_copy(iid_hbm.at[wid], iidx)
        pltpu.sync_copy(gb_hbm, gbv)

        copies = []
        for c in range(NCH):
            dst = pl.ds(c * CH, CH)
            copies.append(pltpu.async_copy(
                uemb_hbm.at[uidx.at[c]], urows.at[dst], sem_u))
            copies.append(pltpu.async_copy(
                iemb_hbm.at[iidx.at[c]], irows.at[dst], sem_i))
            copies.append(pltpu.async_copy(
                ub_hbm.at[uidx.at[c]], ubv.at[dst], sem_ub))
            copies.append(pltpu.async_copy(
                ib_hbm.at[iidx.at[c]], ibv.at[dst], sem_ib))
        for cp_ in copies:
            cp_.wait()

        gb = gbv[...]
        lane = lax.iota(jnp.int32, L)

        @pl.loop(0, BW, step=L)
        def _(t):
            vec = jnp.zeros((L,), jnp.float32)
            for r in range(L):
                b = t + r
                u0 = urows[b, pl.ds(0, L)]
                u1 = urows[b, pl.ds(L, L)]
                v0 = irows[b, pl.ds(0, L)]
                v1 = irows[b, pl.ds(L, L)]
                s = jnp.sum(u0 * v0 + u1 * v1)
                vec = jnp.where(lane == r, s, vec)
            d = pl.ds(t, L)
            outv[d] = vec + ubv[d] + ibv[d] + gb

        pltpu.sync_copy(outv, out_hbm.at[pl.ds(wid * BW, BW)])

    return k(uid, iid, user_emb, item_emb, ub_flat, ib_flat, global_bias)


def kernel(user_ids, item_ids, user_emb, item_emb, user_bias, item_bias,
           global_bias):
    uid = user_ids.reshape(NW, NCH, CH)
    iid = item_ids.reshape(NW, NCH, CH)
    ub = user_bias.reshape(-1)
    ib = item_bias.reshape(-1)
    gbl = jnp.broadcast_to(global_bias.reshape(()), (L,))
    return _sc_predict(uid, iid, user_emb, item_emb, ub, ib, gbl)
